# baseline (speedup 1.0000x reference)
import functools

import jax
import jax.numpy as jnp
from jax import lax
from jax.experimental import pallas as pl
from jax.experimental.pallas import tpu as pltpu
from jax.experimental.pallas import tpu_sc as plsc

N = 10000
NP = 10240
E = 160000
H = 2
C = 325
DP = 656
NEG_SLOPE = 0.2
NEG = -3.0e38

NTILES = 32
CHUNK1 = 2000
T1W = 144

_i16 = lambda: lax.iota(jnp.int32, 16)


def _splat(buf, k):
    return plsc.load_gather(buf, [jnp.full((16,), 0, jnp.int32) + k])


def _tca_body(h_ref, w_ref, o_ref):
    hb = h_ref[...]
    y = jnp.dot(hb, w_ref[...], preferred_element_type=jnp.float32)
    h5 = hb[:, :5]
    o_ref[:, 0:5] = h5
    o_ref[:, 5:10] = h5
    o_ref[:, 10:11] = jnp.ones_like(hb[:, :1])
    o_ref[:, 11:16] = jnp.zeros_like(h5)
    o_ref[:, 16:144] = y


def _tc_a(h8, wcat8):
    return pl.pallas_call(
        _tca_body,
        grid=(8,),
        in_specs=[
            pl.BlockSpec((1280, 8), lambda i: (i, 0)),
            pl.BlockSpec((8, 128), lambda i: (0, 0)),
        ],
        out_specs=pl.BlockSpec((1280, T1W), lambda i: (i, 0)),
        out_shape=jax.ShapeDtypeStruct((NP, T1W), jnp.float32),
    )(h8, wcat8)


def _ewsum_body(ew_ref, o_ref):
    o_ref[...] = jnp.sum(ew_ref[...]).reshape(1, 1)


def _ew_sum(ew2d):
    return pl.pallas_call(
        _ewsum_body,
        out_shape=jax.ShapeDtypeStruct((1, 1), jnp.float32),
    )(ew2d)


def _sc1_body(t1, src, dst, ew, sums_o, m3_o, m5_o,
              dstb, srcb, ewb, accs, acc3, acc5, grow, mb, semg):
    wid = lax.axis_index("s") * 2 + lax.axis_index("c")
    lo = wid * 320
    it = _i16()

    def _init(i, _):
        accs[pl.ds(i * 16, 16)] = jnp.zeros((16,), jnp.float32)
        return 0
    lax.fori_loop(0, 320, _init, 0)

    def _init2(i, _):
        acc3[pl.ds(i * 16, 16)] = jnp.full((16,), NEG, jnp.float32)
        acc5[pl.ds(i * 16, 16)] = jnp.full((16,), NEG, jnp.float32)
        return 0
    lax.fori_loop(0, 1280, _init2, 0)

    wsel_base = jnp.where(it < 5, 0.0, 1.0)

    def _chunk(g, _):
        off = g * CHUNK1
        pltpu.sync_copy(dst.at[pl.ds(off, CHUNK1)], dstb)
        pltpu.sync_copy(src.at[pl.ds(off, CHUNK1)], srcb)
        pltpu.sync_copy(ew.at[pl.ds(off, CHUNK1)], ewb)

        def _scan(j, carry):
            v = dstb[pl.ds(j * 16, 16)]
            m = (v >= lo) & (v < lo + 320)
            nmg = jnp.sum(m.astype(jnp.int32))

            mi32 = m.astype(jnp.int32)

            @pl.when(nmg > 0)
            def _():
                vsrc = srcb[pl.ds(j * 16, 16)]
                vew = ewb[pl.ds(j * 16, 16)]
                pltpu.async_copy(t1.at[jnp.where(m, vsrc, 0)], grow, semg).wait()
                for kk in range(16):
                    @pl.when(mi32[kk] == 1)
                    def _():
                        dv = v[kk] - lo
                        wv = vew[kk]
                        hrow = plsc.load_gather(grow, [jnp.full((16,), kk, jnp.int32), it])
                        wmul = jnp.where(it < 5, wv, wsel_base)
                        plsc.addupdate(accs.at[pl.ds(dv * 16, 16)], hrow * wmul)
                        for jj in range(4):
                            cidx = 16 + jj * 16 + it
                            r3 = plsc.load_gather(grow, [jnp.full((16,), kk, jnp.int32), cidx])
                            b3 = dv * 64 + jj * 16
                            acc3[pl.ds(b3, 16)] = jnp.maximum(acc3[pl.ds(b3, 16)], wv * r3)
                        for jj in range(4):
                            cidx = 80 + jj * 16 + it
                            r5 = plsc.load_gather(grow, [jnp.full((16,), kk, jnp.int32), cidx])
                            b5 = dv * 64 + jj * 16
                            acc5[pl.ds(b5, 16)] = jnp.maximum(acc5[pl.ds(b5, 16)], r5)
            return carry
        lax.fori_loop(0, CHUNK1 // 16, _scan, 0)
        return 0

    lax.fori_loop(0, E // CHUNK1, _chunk, 0)

    pltpu.sync_copy(accs, sums_o.at[pl.ds(lo * 16, 320 * 16)])
    pltpu.sync_copy(acc3, m3_o.at[pl.ds(lo * 64, 320 * 64)])
    pltpu.sync_copy(acc5, m5_o.at[pl.ds(lo * 64, 320 * 64)])


def _sc_1(t1, src, dst, ew):
    mesh = plsc.VectorSubcoreMesh(core_axis_name="c", subcore_axis_name="s")
    f = pl.kernel(
        _sc1_body,
        compiler_params=pltpu.CompilerParams(
            needs_layout_passes=False, use_tc_tiling_on_sc=False),
        out_type=[
            jax.ShapeDtypeStruct((NP * 16,), jnp.float32),
            jax.ShapeDtypeStruct((NP * 64,), jnp.float32),
            jax.ShapeDtypeStruct((NP * 64,), jnp.float32),
        ],
        mesh=mesh,
        scratch_types=[
            pltpu.VMEM((CHUNK1,), jnp.int32),
            pltpu.VMEM((CHUNK1,), jnp.int32),
            pltpu.VMEM((CHUNK1,), jnp.float32),
            pltpu.VMEM((320 * 16,), jnp.float32),
            pltpu.VMEM((320 * 64,), jnp.float32),
            pltpu.VMEM((320 * 64,), jnp.float32),
            pltpu.VMEM((16, T1W), jnp.float32),
            pltpu.VMEM((16,), jnp.int32),
            pltpu.SemaphoreType.DMA,
        ],
    )
    return f(t1, src, dst, ew)


def kernel(h, edge_num, edge_index, edge_weight, W_sum, b_sum, W_mean, b_mean,
           W_max, b_max, W_ne, b_ne, W_nem, b_nem, Wl, bl, Wr, br, We, att,
           gat_bias, Wfc, bfc):
    src = edge_index[0]
    dst = edge_index[1]

    h8 = jnp.pad(h, ((0, NP - N), (0, 3)))
    wcat8 = jnp.pad(jnp.concatenate([W_max, W_nem], axis=1), ((0, 3), (0, 0)))
    t1 = _tc_a(h8, wcat8)
    ewsum = _ew_sum(edge_weight.reshape(1250, 128))
    ewm = ewsum[0, 0] / E

    sums_f, m3_f, m5_f = _sc_1(t1, src, dst, edge_weight)
    sums = sums_f.reshape(NP, 16)[:N]
    M3 = m3_f.reshape(NP, 64)[:N]
    M5 = m5_f.reshape(NP, 64)[:N]

    S1 = sums[:, 0:5]
    S2 = sums[:, 5:10]
    cnt = sums[:, 10:11]
    has = cnt > 0

    h1 = S1 @ W_sum + b_sum
    h2 = (S1 @ W_mean) / jnp.maximum(cnt, 1.0) + b_mean
    h3 = jnp.where(has, M3, 0.0) + b_max
    h4 = S2 @ W_ne + b_ne
    h5 = jnp.where(has, M5, 0.0) + b_nem
    x = jnp.concatenate([h1, h2, h3, h4, h5, edge_num], axis=-1)

    loop = jnp.arange(N, dtype=src.dtype)
    src2 = jnp.concatenate([src, loop])
    dst2 = jnp.concatenate([dst, loop])
    ea = jnp.concatenate([edge_weight, jnp.full((N,), ewm, dtype=edge_weight.dtype)])[:, None]
    xl = (x @ Wl + bl).reshape(N, H, C)
    xr = (x @ Wr + br).reshape(N, H, C)
    x_j = xl[src2]
    x_i = xr[dst2]
    e = (ea @ We).reshape(-1, H, C)
    m = jax.nn.leaky_relu(x_i + x_j + e, NEG_SLOPE)
    alpha = jnp.sum(m * att[None, :, :], axis=-1)
    amax = jax.ops.segment_max(alpha, dst2, num_segments=N)
    amax = jnp.where(jnp.isfinite(amax), amax, 0.0)
    alpha = jnp.exp(alpha - amax[dst2])
    denom = jax.ops.segment_sum(alpha, dst2, num_segments=N)
    alpha = alpha / (denom[dst2] + 1e-16)
    msg = x_j * alpha[:, :, None]
    s = jax.ops.segment_sum(msg, dst2, num_segments=N)
    cnt2 = jax.ops.segment_sum(jnp.ones((msg.shape[0],), msg.dtype), dst2, num_segments=N)
    out = s / jnp.maximum(cnt2, 1.0)[:, None, None]
    out = out.mean(axis=1) + gat_bias
    return out @ Wfc + bfc
